# baseline (speedup 1.0000x reference)
import functools

import jax
import jax.numpy as jnp
from jax import lax
from jax.experimental import pallas as pl
from jax.experimental.pallas import tpu as pltpu
from jax.experimental.pallas import tpu_sc as plsc

N = 10000
MAX_DEG = 32
D = 128

NC = 2
NS = 16
NW = NC * NS

PAD_N = 10240
PER_W = PAD_N // NW
C = 16
NB = PER_W // C
UW = 256
NU = PER_W * MAX_DEG // UW
NSUPER = NU // 2
SENT = -1


def _sc_sum_body(table_hbm, idxrm_hbm, vl_hbm, out_hbm,
                 idx_all, dst_all, vl_all, rows_v, zero_v, acc_sh,
                 sem_g, sem_s):
    cc = lax.axis_index("c")
    ss = lax.axis_index("s")
    wid = ss * NC + cc
    abase = ss * PER_W
    obase = wid * PER_W

    pltpu.sync_copy(idxrm_hbm.at[pl.ds(wid * NU * UW, NU * UW)], idx_all)
    pltpu.sync_copy(vl_hbm.at[pl.ds(obase, PER_W)], vl_all)

    zv = jnp.zeros((16,), jnp.float32)
    for r in range(C):
        for t in range(D // 16):
            zero_v[r, pl.ds(t * 16, 16)] = zv

    def zero_body(z, carry):
        pltpu.sync_copy(zero_v, acc_sh.at[pl.ds(abase + z * C, C)])
        return carry
    lax.fori_loop(0, NB, zero_body, 0)

    iota16 = jnp.arange(16, dtype=jnp.int32)
    sent = jnp.full((16,), SENT, jnp.int32)

    def fix_body(q, carry):
        vl16 = vl_all[pl.ds((q // 2) * 16, 16)]
        for t in range(16):
            n = 8 * q + t // 2
            loc = 8 * (q % 2) + t // 2
            vln = vl16.at[jnp.full((16,), loc, jnp.int32)].get(
                mode="promise_in_bounds")
            jvec = iota16 + (t % 2) * 16
            m = jvec < vln
            iv = idx_all[pl.ds(q * UW + t * 16, 16)]
            idx_all[pl.ds(q * UW + t * 16, 16)] = jnp.where(m, iv, sent)
            dst_all[pl.ds(q * UW + t * 16, 16)] = jnp.where(
                m, jnp.full((16,), abase + n, jnp.int32), sent)
        return carry
    lax.fori_loop(0, NU, fix_body, 0)

    def _gsrc(u):
        return table_hbm.at[plsc.Indices(idx_all.at[pl.ds(u * UW, UW)],
                                         ignored_value=SENT)]

    def _sdst(u):
        return acc_sh.at[plsc.Indices(dst_all.at[pl.ds(u * UW, UW)],
                                      ignored_value=SENT)]

    def _buf(p):
        return rows_v.at[pl.ds(p * 256, 256)]

    def gfire(u, p):
        pltpu.async_copy(_gsrc(u), _buf(p), sem_g)

    def gwait(u, p):
        pltpu.make_async_copy(_gsrc(u), _buf(p), sem_g).wait()

    def sfire(u, p):
        pltpu.async_copy(_buf(p), _sdst(u), sem_s, add=True)

    def swait(u, p):
        pltpu.make_async_copy(_buf(p), _sdst(u), sem_s).wait()

    def super_body(s, carry):
        u = 2 * s

        @pl.when(s > 0)
        def _free0():
            swait(u - 2, 0)
        gfire(u, 0)

        @pl.when(s > 0)
        def _free1():
            swait(u - 1, 1)
        gfire(u + 1, 1)
        gwait(u, 0)
        sfire(u, 0)
        gwait(u + 1, 1)
        sfire(u + 1, 1)
        return carry

    lax.fori_loop(0, NSUPER, super_body, 0)

    ulast = NU - 2
    swait(ulast, 0)
    swait(ulast + 1, 1)

    pltpu.sync_copy(acc_sh.at[pl.ds(abase, PER_W)],
                    out_hbm.at[pl.ds(obase, PER_W)])


_sc_sum = functools.partial(
    pl.kernel,
    out_type=jax.ShapeDtypeStruct((PAD_N, D), jnp.float32),
    mesh=plsc.VectorSubcoreMesh(core_axis_name="c", subcore_axis_name="s"),
    scratch_types=[
        pltpu.VMEM((NU * UW,), jnp.int32),
        pltpu.VMEM((NU * UW,), jnp.int32),
        pltpu.VMEM((PER_W,), jnp.int32),
        pltpu.VMEM((512, D), jnp.float32),
        pltpu.VMEM((C, D), jnp.float32),
        pltpu.VMEM_SHARED((NS * PER_W, D), jnp.float32),
        pltpu.SemaphoreType.DMA,
        pltpu.SemaphoreType.DMA,
    ],
)(_sc_sum_body)


def _tc_dense_body(s_ref, v_ref, vl_ref, w_ref, b_ref, o_ref):
    vlf = jnp.maximum(vl_ref[...], 1).astype(jnp.float32)
    mean = s_ref[...] / vlf
    o = (jnp.dot(mean, w_ref[...], preferred_element_type=jnp.float32)
         + jnp.dot(v_ref[...], b_ref[...], preferred_element_type=jnp.float32))
    o_ref[...] = jnp.maximum(o, 0.0)


def _tc_dense(sums, vertex_feat, vl2d, W, B):
    R = 1000
    grid = (N // R,)
    return pl.pallas_call(
        _tc_dense_body,
        grid=grid,
        in_specs=[
            pl.BlockSpec((R, D), lambda i: (i, 0)),
            pl.BlockSpec((R, D), lambda i: (i, 0)),
            pl.BlockSpec((R, 1), lambda i: (i, 0)),
            pl.BlockSpec((D, D), lambda i: (0, 0)),
            pl.BlockSpec((D, D), lambda i: (0, 0)),
        ],
        out_specs=pl.BlockSpec((R, D), lambda i: (i, 0)),
        out_shape=jax.ShapeDtypeStruct((N, D), jnp.float32),
    )(sums, vertex_feat, vl2d, W, B)


def kernel(vertex_feat, neighbors_idx, valid_lens, W, B):
    table = vertex_feat
    idx_rm = jnp.pad(neighbors_idx.reshape(N * MAX_DEG),
                     (0, (PAD_N - N) * MAX_DEG))
    vl_p = jnp.zeros((PAD_N,), jnp.int32).at[:N].set(valid_lens)
    sums = _sc_sum(table, idx_rm, vl_p)
    return _tc_dense(sums, vertex_feat, valid_lens.reshape(N, 1), W, B)


# speedup vs baseline: 1.0216x; 1.0216x over previous
import functools

import jax
import jax.numpy as jnp
from jax import lax
from jax.experimental import pallas as pl
from jax.experimental.pallas import tpu as pltpu
from jax.experimental.pallas import tpu_sc as plsc

N = 10000
MAX_DEG = 32
D = 128

NC = 2
NS = 16
NW = NC * NS

PAD_N = 10240
PER_W = PAD_N // NW
C = 16
NB = PER_W // C
QPB = C * MAX_DEG // 128
NQ = NB * QPB
NSUPER = NQ // 4
SENT = -1


def _sc_sum_body(table_hbm, idxrm_hbm, vl_hbm, out_hbm,
                 idx_all, dst_all, vl_all, rows_v, zero_v, acc_sh,
                 sem_g, sem_s):
    cc = lax.axis_index("c")
    ss = lax.axis_index("s")
    wid = ss * NC + cc
    abase = ss * PER_W
    obase = wid * PER_W

    pltpu.sync_copy(idxrm_hbm.at[pl.ds(wid * NQ, NQ)], idx_all)
    pltpu.sync_copy(vl_hbm.at[pl.ds(obase, PER_W)], vl_all)

    zv = jnp.zeros((16,), jnp.float32)
    for r in range(C):
        for t in range(D // 16):
            zero_v[r, pl.ds(t * 16, 16)] = zv

    def zero_body(z, carry):
        pltpu.sync_copy(zero_v, acc_sh.at[pl.ds(abase + z * C, C)])
        return carry
    lax.fori_loop(0, NB, zero_body, 0)

    iota16 = jnp.arange(16, dtype=jnp.int32)
    sent = jnp.full((16,), SENT, jnp.int32)

    def fix_body(q, carry):
        vl16 = vl_all[pl.ds((q // 4) * 16, 16)]
        for t in range(8):
            n = 4 * q + t // 2
            loc = 4 * (q % 4) + t // 2
            vln = vl16.at[jnp.full((16,), loc, jnp.int32)].get(
                mode="promise_in_bounds")
            jvec = iota16 + (t % 2) * 16
            m = jvec < vln
            iv = idx_all[q, pl.ds(t * 16, 16)]
            idx_all[q, pl.ds(t * 16, 16)] = jnp.where(m, iv, sent)
            dst_all[q, pl.ds(t * 16, 16)] = jnp.where(
                m, jnp.full((16,), abase + n, jnp.int32), sent)
        return carry
    lax.fori_loop(0, NQ, fix_body, 0)

    def _gsrc(u):
        return table_hbm.at[plsc.Indices(idx_all.at[u], ignored_value=SENT)]

    def _sdst(u):
        return acc_sh.at[plsc.Indices(dst_all.at[u], ignored_value=SENT)]

    def _buf(p):
        return rows_v.at[pl.ds(p * 128, 128)]

    def gfire(u, p):
        pltpu.async_copy(_gsrc(u), _buf(p), sem_g)

    def gwait(u, p):
        pltpu.make_async_copy(_gsrc(u), _buf(p), sem_g).wait()

    def sfire(u, p):
        pltpu.async_copy(_buf(p), _sdst(u), sem_s, add=True)

    def swait(u, p):
        pltpu.make_async_copy(_buf(p), _sdst(u), sem_s).wait()

    def super_body(s, carry):
        u = 4 * s

        @pl.when(s > 0)
        def _free01():
            swait(u - 4, 0)
            swait(u - 3, 1)
        gfire(u, 0)
        gfire(u + 1, 1)

        @pl.when(s > 0)
        def _free23():
            swait(u - 2, 2)
            swait(u - 1, 3)
        gfire(u + 2, 2)
        gfire(u + 3, 3)
        for p in range(4):
            gwait(u + p, p)
            sfire(u + p, p)
        return carry

    lax.fori_loop(0, NSUPER, super_body, 0)

    ulast = NQ - 4
    for p in range(4):
        swait(ulast + p, p)

    pltpu.sync_copy(acc_sh.at[pl.ds(abase, PER_W)],
                    out_hbm.at[pl.ds(obase, PER_W)])


_sc_sum = functools.partial(
    pl.kernel,
    out_type=jax.ShapeDtypeStruct((PAD_N, D), jnp.float32),
    mesh=plsc.VectorSubcoreMesh(core_axis_name="c", subcore_axis_name="s"),
    scratch_types=[
        pltpu.VMEM((NQ, 128), jnp.int32),
        pltpu.VMEM((NQ, 128), jnp.int32),
        pltpu.VMEM((PER_W,), jnp.int32),
        pltpu.VMEM((512, D), jnp.float32),
        pltpu.VMEM((C, D), jnp.float32),
        pltpu.VMEM_SHARED((NS * PER_W, D), jnp.float32),
        pltpu.SemaphoreType.DMA,
        pltpu.SemaphoreType.DMA,
    ],
)(_sc_sum_body)


def _tc_dense_body(s_ref, v_ref, vl_ref, w_ref, b_ref, o_ref):
    vlf = jnp.maximum(vl_ref[...], 1).astype(jnp.float32)
    mean = s_ref[...] / vlf
    o = (jnp.dot(mean, w_ref[...], preferred_element_type=jnp.float32)
         + jnp.dot(v_ref[...], b_ref[...], preferred_element_type=jnp.float32))
    o_ref[...] = jnp.maximum(o, 0.0)


def _tc_dense(sums, vertex_feat, vl2d, W, B):
    R = 1000
    grid = (N // R,)
    return pl.pallas_call(
        _tc_dense_body,
        grid=grid,
        in_specs=[
            pl.BlockSpec((R, D), lambda i: (i, 0)),
            pl.BlockSpec((R, D), lambda i: (i, 0)),
            pl.BlockSpec((R, 1), lambda i: (i, 0)),
            pl.BlockSpec((D, D), lambda i: (0, 0)),
            pl.BlockSpec((D, D), lambda i: (0, 0)),
        ],
        out_specs=pl.BlockSpec((R, D), lambda i: (i, 0)),
        out_shape=jax.ShapeDtypeStruct((N, D), jnp.float32),
    )(sums, vertex_feat, vl2d, W, B)


def kernel(vertex_feat, neighbors_idx, valid_lens, W, B):
    table = vertex_feat
    idx_rm = jnp.pad(neighbors_idx.reshape(N * MAX_DEG // 128, 128),
                     ((0, (PAD_N - N) * MAX_DEG // 128), (0, 0)))
    vl_p = jnp.zeros((PAD_N,), jnp.int32).at[:N].set(valid_lens)
    sums = _sc_sum(table, idx_rm, vl_p)
    return _tc_dense(sums, vertex_feat, valid_lens.reshape(N, 1), W, B)
